# baseline (speedup 1.0000x reference)
import functools

import jax
import jax.numpy as jnp
from jax.experimental import pallas as pl

NA = 32
NACT = 8
DPRE = 64
GPB = 4
R = NA * GPB


def _critic_kernel(obs_ref, z_ref, pol_ref, act_ref,
                   wfc_ref, bfc_ref, w1_ref, b1_ref, w2_ref, b2_ref,
                   wv_ref, bv_ref, x_ref, w_ref):
    f32 = jnp.float32
    dot = functools.partial(jax.lax.dot_general,
                            preferred_element_type=f32)
    mm = lambda a, b: dot(a, b, (((1,), (0,)), ((), ())))
    mm_t = lambda a, b: dot(a, b, (((1,), (1,)), ((), ())))

    row_g = jax.lax.broadcasted_iota(jnp.int32, (R, R), 0) // NA
    col_g = jax.lax.broadcasted_iota(jnp.int32, (R, R), 1) // NA
    mask = (row_g == col_g).astype(f32)
    sel_c = jax.lax.broadcasted_iota(jnp.int32, (R, NA), 0) % NA
    sel_j = jax.lax.broadcasted_iota(jnp.int32, (R, NA), 1)
    T = (sel_c == sel_j).astype(f32)

    w1a = w1_ref[0:1, :DPRE]
    w1b = w1_ref[0:1, DPRE:]
    w2a = w2_ref[0:1, :NA]
    w2b = w2_ref[0:1, NA:]
    wv1 = wv_ref[0:1, :DPRE]
    wv2 = wv_ref[0:1, DPRE:DPRE + NACT]
    b1 = b1_ref[0]
    b2 = b2_ref[0]
    bv = bv_ref[0]

    obs = obs_ref[...]
    F = mm_t(obs, wfc_ref[...]) + bfc_ref[...]

    a_row = mm_t(w1a, F)
    c_col = mm_t(F, w1b)
    alpha = jax.nn.sigmoid(c_col + a_row + b1) * mask
    obs_proc = mm(alpha, F)
    s_col = mm_t(obs_proc, wv1)

    z = z_ref[...]
    u_row = mm_t(w2a, z)
    v_col = mm_t(z, w2b)
    wfull = jax.nn.sigmoid(v_col + u_row + b2) * mask

    pol = pol_ref[...]
    dp_col = mm_t(act_ref[...] - pol, wv2)
    pp_col = mm_t(pol, wv2)

    rhs_w = jnp.concatenate([T, dp_col * (1.0 / NA)], axis=1)
    ws = mm(wfull, rhs_w)
    w_out = ws[:, :NA]
    S_col = ws[:, NA:NA + 1]
    rhs_m = jnp.concatenate([T * s_col, T * dp_col, pp_col * (1.0 / NA)],
                            axis=1)
    rows = mm(mask, rhs_m)
    s_rows = rows[:, :NA]
    dp_rows = rows[:, NA:2 * NA]
    pm_col = rows[:, 2 * NA:2 * NA + 1]

    w_ref[...] = w_out
    x_ref[...] = (S_col + pm_col + bv) + s_rows \
        - w_out * dp_rows * (1.0 / NA)


def kernel(obs, mypose_goalpose, policies, actions,
           W_fc, b_fc, W_attn_in, b_attn_in, W_attn_w, b_attn_w,
           W_val, b_val):
    n = obs.shape[0]
    grid = n // R

    row_spec = lambda w: pl.BlockSpec((R, w), lambda i: (i, 0))
    full = lambda a: pl.BlockSpec(a.shape, lambda *_: (0,) * a.ndim)

    x2d, w2d = pl.pallas_call(
        _critic_kernel,
        grid=(grid,),
        in_specs=[
            row_spec(obs.shape[1]),
            row_spec(mypose_goalpose.shape[1]),
            row_spec(NACT),
            row_spec(NACT),
            full(W_fc), full(b_fc),
            full(W_attn_in), full(b_attn_in),
            full(W_attn_w), full(b_attn_w),
            full(W_val), full(b_val),
        ],
        out_specs=[row_spec(NA), row_spec(NA)],
        out_shape=[
            jax.ShapeDtypeStruct((n, NA), jnp.float32),
            jax.ShapeDtypeStruct((n, NA), jnp.float32),
        ],
    )(obs, mypose_goalpose, policies, actions,
      W_fc, b_fc, W_attn_in, b_attn_in, W_attn_w, b_attn_w, W_val, b_val)

    return x2d[:, :, None], w2d[:, :, None]


# speedup vs baseline: 1.4175x; 1.4175x over previous
import functools

import jax
import jax.numpy as jnp
from jax.experimental import pallas as pl

NA = 32
NACT = 8
DPRE = 64
GPB = 16
R = NA * GPB


def _critic_kernel(obs_ref, z_ref, pol_ref, act_ref,
                   wfc_ref, bfc_ref, w1_ref, b1_ref, w2_ref, b2_ref,
                   wv_ref, bv_ref, x_ref, w_ref):
    f32 = jnp.float32
    dot = functools.partial(jax.lax.dot_general,
                            preferred_element_type=f32)
    mm = lambda a, b: dot(a, b, (((1,), (0,)), ((), ())))
    mm_t = lambda a, b: dot(a, b, (((1,), (1,)), ((), ())))

    row_g = jax.lax.broadcasted_iota(jnp.int32, (R, R), 0) // NA
    col_g = jax.lax.broadcasted_iota(jnp.int32, (R, R), 1) // NA
    mask = (row_g == col_g).astype(f32)
    sel_c = jax.lax.broadcasted_iota(jnp.int32, (R, NA), 0) % NA
    sel_j = jax.lax.broadcasted_iota(jnp.int32, (R, NA), 1)
    T = (sel_c == sel_j).astype(f32)

    w1a = w1_ref[0:1, :DPRE]
    w1b = w1_ref[0:1, DPRE:]
    w2a = w2_ref[0:1, :NA]
    w2b = w2_ref[0:1, NA:]
    wv1 = wv_ref[0:1, :DPRE]
    wv2 = wv_ref[0:1, DPRE:DPRE + NACT]
    b1 = b1_ref[0]
    b2 = b2_ref[0]
    bv = bv_ref[0]

    obs = obs_ref[...]
    F = mm_t(obs, wfc_ref[...]) + bfc_ref[...]

    a_row = mm_t(w1a, F)
    c_col = mm_t(F, w1b)
    alpha = jax.nn.sigmoid(c_col + a_row + b1) * mask
    obs_proc = mm(alpha, F)
    s_col = mm_t(obs_proc, wv1)

    z = z_ref[...]
    u_row = mm_t(w2a, z)
    v_col = mm_t(z, w2b)
    wfull = jax.nn.sigmoid(v_col + u_row + b2) * mask

    pol = pol_ref[...]
    dp_col = mm_t(act_ref[...] - pol, wv2)
    pp_col = mm_t(pol, wv2)

    rhs_w = jnp.concatenate([T, dp_col * (1.0 / NA)], axis=1)
    ws = mm(wfull, rhs_w)
    w_out = ws[:, :NA]
    S_col = ws[:, NA:NA + 1]
    rhs_m = jnp.concatenate([T * s_col, T * dp_col, pp_col * (1.0 / NA)],
                            axis=1)
    rows = mm(mask, rhs_m)
    s_rows = rows[:, :NA]
    dp_rows = rows[:, NA:2 * NA]
    pm_col = rows[:, 2 * NA:2 * NA + 1]

    w_ref[...] = w_out
    x_ref[...] = (S_col + pm_col + bv) + s_rows \
        - w_out * dp_rows * (1.0 / NA)


def kernel(obs, mypose_goalpose, policies, actions,
           W_fc, b_fc, W_attn_in, b_attn_in, W_attn_w, b_attn_w,
           W_val, b_val):
    n = obs.shape[0]
    grid = n // R

    row_spec = lambda w: pl.BlockSpec((R, w), lambda i: (i, 0))
    full = lambda a: pl.BlockSpec(a.shape, lambda *_: (0,) * a.ndim)

    x2d, w2d = pl.pallas_call(
        _critic_kernel,
        grid=(grid,),
        in_specs=[
            row_spec(obs.shape[1]),
            row_spec(mypose_goalpose.shape[1]),
            row_spec(NACT),
            row_spec(NACT),
            full(W_fc), full(b_fc),
            full(W_attn_in), full(b_attn_in),
            full(W_attn_w), full(b_attn_w),
            full(W_val), full(b_val),
        ],
        out_specs=[row_spec(NA), row_spec(NA)],
        out_shape=[
            jax.ShapeDtypeStruct((n, NA), jnp.float32),
            jax.ShapeDtypeStruct((n, NA), jnp.float32),
        ],
    )(obs, mypose_goalpose, policies, actions,
      W_fc, b_fc, W_attn_in, b_attn_in, W_attn_w, b_attn_w, W_val, b_val)

    return x2d[:, :, None], w2d[:, :, None]
